# initial kernel scaffold (unmeasured)
import jax
import jax.numpy as jnp
from jax import lax
from jax.experimental import pallas as pl
from jax.experimental.pallas import tpu as pltpu

N_DEV = 32
M_PER = 1024
K = 512
N_OUT = 512

RS_XOR = [1, 8, 2, 4, 16]


def kernel(t, W):
    def body(t_ref, w_ref, out_ref, acc_ref, rbuf_ref, send_sems, recv_sems):
        my = lax.axis_index("i")

        barrier_sem = pltpu.get_barrier_semaphore()
        for d in RS_XOR:
            pl.semaphore_signal(
                barrier_sem, inc=1,
                device_id=(my ^ d,), device_id_type=pl.DeviceIdType.MESH,
            )
        pl.semaphore_wait(barrier_sem, len(RS_XOR))

        acc_ref[:, :] = t_ref[:, :]

        lo = jnp.int32(0)
        size = M_PER
        for r, d in enumerate(RS_XOR):
            half = size // 2
            shift = d.bit_length() - 1
            bit = (my >> shift) & 1
            partner = my ^ d
            send_lo = lo + (1 - bit) * half
            keep_lo = lo + bit * half
            rdma = pltpu.make_async_remote_copy(
                src_ref=acc_ref.at[pl.ds(send_lo, half)],
                dst_ref=rbuf_ref.at[pl.ds(0, half)],
                send_sem=send_sems.at[r],
                recv_sem=recv_sems.at[r],
                device_id=(partner,),
                device_id_type=pl.DeviceIdType.MESH,
            )
            rdma.start()
            rdma.wait()
            acc_ref[pl.ds(keep_lo, half), :] = (
                acc_ref[pl.ds(keep_lo, half), :] + rbuf_ref[pl.ds(0, half), :]
            )
            lo = keep_lo
            size = half

        out_ref[pl.ds(lo, size), :] = jnp.dot(
            acc_ref[pl.ds(lo, size), :], w_ref[:, :],
            preferred_element_type=jnp.float32,
        )

        for j, d in enumerate(reversed(RS_XOR)):
            r = len(RS_XOR) + j
            shift = d.bit_length() - 1
            bit = (my >> shift) & 1
            partner = my ^ d
            rdma = pltpu.make_async_remote_copy(
                src_ref=out_ref.at[pl.ds(lo, size)],
                dst_ref=out_ref.at[pl.ds(lo, size)],
                send_sem=send_sems.at[r],
                recv_sem=recv_sems.at[r],
                device_id=(partner,),
                device_id_type=pl.DeviceIdType.MESH,
            )
            rdma.start()
            rdma.wait()
            lo = lo - bit * size
            size = size * 2

    return pl.pallas_call(
        body,
        out_shape=jax.ShapeDtypeStruct((M_PER, N_OUT), jnp.float32),
        in_specs=[
            pl.BlockSpec(memory_space=pltpu.VMEM),
            pl.BlockSpec(memory_space=pltpu.VMEM),
        ],
        out_specs=pl.BlockSpec(memory_space=pltpu.VMEM),
        scratch_shapes=[
            pltpu.VMEM((M_PER, K), jnp.float32),
            pltpu.VMEM((M_PER // 2, K), jnp.float32),
            pltpu.SemaphoreType.DMA((2 * len(RS_XOR),)),
            pltpu.SemaphoreType.DMA((2 * len(RS_XOR),)),
        ],
        compiler_params=pltpu.CompilerParams(collective_id=0),
    )(t, W)


# baseline (device time: 72977 ns/iter reference)
import jax
import jax.numpy as jnp
from jax import lax
from jax.experimental import pallas as pl
from jax.experimental.pallas import tpu as pltpu

N_DEV = 32
M_PER = 1024
K = 512
N_OUT = 512

RS_XOR = [1, 8, 2, 4, 16]


def kernel(t, W):
    def body(t_ref, w_ref, out_ref, acc_ref, rbuf_ref, send_sems, recv_sems):
        my = lax.axis_index("i")

        barrier_sem = pltpu.get_barrier_semaphore()
        for d in RS_XOR:
            pl.semaphore_signal(
                barrier_sem, inc=1,
                device_id=(my ^ d,), device_id_type=pl.DeviceIdType.MESH,
            )
        pl.semaphore_wait(barrier_sem, len(RS_XOR))

        acc_ref[:, :] = t_ref[:, :]

        lo = jnp.int32(0)
        size = M_PER
        rb_off = 0
        for r, d in enumerate(RS_XOR):
            half = size // 2
            shift = d.bit_length() - 1
            bit = (my >> shift) & 1
            partner = my ^ d
            send_lo = lo + (1 - bit) * half
            keep_lo = lo + bit * half
            rdma = pltpu.make_async_remote_copy(
                src_ref=acc_ref.at[pl.ds(send_lo, half)],
                dst_ref=rbuf_ref.at[pl.ds(rb_off, half)],
                send_sem=send_sems.at[r],
                recv_sem=recv_sems.at[r],
                device_id=(partner,),
                device_id_type=pl.DeviceIdType.MESH,
            )
            rdma.start()
            rdma.wait()
            acc_ref[pl.ds(keep_lo, half), :] = (
                acc_ref[pl.ds(keep_lo, half), :]
                + rbuf_ref[pl.ds(rb_off, half), :]
            )
            lo = keep_lo
            size = half
            rb_off += half

        out_ref[pl.ds(lo, size), :] = jnp.dot(
            acc_ref[pl.ds(lo, size), :], w_ref[:, :],
            preferred_element_type=jnp.float32,
        )

        for j, d in enumerate(reversed(RS_XOR)):
            r = len(RS_XOR) + j
            shift = d.bit_length() - 1
            bit = (my >> shift) & 1
            partner = my ^ d
            rdma = pltpu.make_async_remote_copy(
                src_ref=out_ref.at[pl.ds(lo, size)],
                dst_ref=out_ref.at[pl.ds(lo, size)],
                send_sem=send_sems.at[r],
                recv_sem=recv_sems.at[r],
                device_id=(partner,),
                device_id_type=pl.DeviceIdType.MESH,
            )
            rdma.start()
            rdma.wait()
            lo = lo - bit * size
            size = size * 2

    return pl.pallas_call(
        body,
        out_shape=jax.ShapeDtypeStruct((M_PER, N_OUT), jnp.float32),
        in_specs=[
            pl.BlockSpec(memory_space=pltpu.VMEM),
            pl.BlockSpec(memory_space=pltpu.VMEM),
        ],
        out_specs=pl.BlockSpec(memory_space=pltpu.VMEM),
        scratch_shapes=[
            pltpu.VMEM((M_PER, K), jnp.float32),
            pltpu.VMEM((M_PER, K), jnp.float32),
            pltpu.SemaphoreType.DMA((2 * len(RS_XOR),)),
            pltpu.SemaphoreType.DMA((2 * len(RS_XOR),)),
        ],
        compiler_params=pltpu.CompilerParams(collective_id=0),
    )(t, W)


# device time: 56108 ns/iter; 1.3007x vs baseline; 1.3007x over previous
import jax
import jax.numpy as jnp
from jax import lax
from jax.experimental import pallas as pl
from jax.experimental.pallas import tpu as pltpu

N_DEV = 32
M_PER = 1024
K = 512
N_OUT = 512

STREAMS = [
    (0, [1, 8, 2, 4, 16]),
    (M_PER // 2, [8, 1, 16, 2, 4]),
]
N_ROUNDS = 5
M_STREAM = M_PER // 2


def kernel(t, W):
    def body(t_ref, w_ref, out_ref, acc_ref, rbuf_ref, send_sems, recv_sems):
        my = lax.axis_index("i")

        barrier_sem = pltpu.get_barrier_semaphore()
        for d in STREAMS[0][1]:
            pl.semaphore_signal(
                barrier_sem, inc=1,
                device_id=(my ^ d,), device_id_type=pl.DeviceIdType.MESH,
            )
        pl.semaphore_wait(barrier_sem, N_ROUNDS)

        acc_ref[:, :] = t_ref[:, :]

        n_s = len(STREAMS)
        los = [jnp.int32(base) for base, _ in STREAMS]
        sizes = [M_STREAM] * n_s
        rb_offs = [base for base, _ in STREAMS]

        for r in range(N_ROUNDS):
            pend = []
            for s, (_, order) in enumerate(STREAMS):
                d = order[r]
                half = sizes[s] // 2
                bit = (my >> (d.bit_length() - 1)) & 1
                send_lo = los[s] + (1 - bit) * half
                keep_lo = los[s] + bit * half
                rdma = pltpu.make_async_remote_copy(
                    src_ref=acc_ref.at[pl.ds(send_lo, half)],
                    dst_ref=rbuf_ref.at[pl.ds(rb_offs[s], half)],
                    send_sem=send_sems.at[s * 2 * N_ROUNDS + r],
                    recv_sem=recv_sems.at[s * 2 * N_ROUNDS + r],
                    device_id=(my ^ d,),
                    device_id_type=pl.DeviceIdType.MESH,
                )
                rdma.start()
                pend.append((rdma, keep_lo, half, rb_offs[s]))
                los[s] = keep_lo
                sizes[s] = half
                rb_offs[s] += half
            for rdma, keep_lo, half, off in pend:
                rdma.wait()
                acc_ref[pl.ds(keep_lo, half), :] = (
                    acc_ref[pl.ds(keep_lo, half), :]
                    + rbuf_ref[pl.ds(off, half), :]
                )

        for s in range(n_s):
            out_ref[pl.ds(los[s], sizes[s]), :] = jnp.dot(
                acc_ref[pl.ds(los[s], sizes[s]), :], w_ref[:, :],
                preferred_element_type=jnp.float32,
            )

        for j in range(N_ROUNDS):
            pend = []
            for s, (_, order) in enumerate(STREAMS):
                d = order[N_ROUNDS - 1 - j]
                bit = (my >> (d.bit_length() - 1)) & 1
                rdma = pltpu.make_async_remote_copy(
                    src_ref=out_ref.at[pl.ds(los[s], sizes[s])],
                    dst_ref=out_ref.at[pl.ds(los[s], sizes[s])],
                    send_sem=send_sems.at[s * 2 * N_ROUNDS + N_ROUNDS + j],
                    recv_sem=recv_sems.at[s * 2 * N_ROUNDS + N_ROUNDS + j],
                    device_id=(my ^ d,),
                    device_id_type=pl.DeviceIdType.MESH,
                )
                rdma.start()
                pend.append((rdma, s, bit))
            for rdma, s, bit in pend:
                rdma.wait()
                los[s] = los[s] - bit * sizes[s]
                sizes[s] = sizes[s] * 2

    n_sems = 2 * N_ROUNDS * len(STREAMS)
    return pl.pallas_call(
        body,
        out_shape=jax.ShapeDtypeStruct((M_PER, N_OUT), jnp.float32),
        in_specs=[
            pl.BlockSpec(memory_space=pltpu.VMEM),
            pl.BlockSpec(memory_space=pltpu.VMEM),
        ],
        out_specs=pl.BlockSpec(memory_space=pltpu.VMEM),
        scratch_shapes=[
            pltpu.VMEM((M_PER, K), jnp.float32),
            pltpu.VMEM((M_PER, K), jnp.float32),
            pltpu.SemaphoreType.DMA((n_sems,)),
            pltpu.SemaphoreType.DMA((n_sems,)),
        ],
        compiler_params=pltpu.CompilerParams(collective_id=0),
    )(t, W)


# device time: 44193 ns/iter; 1.6513x vs baseline; 1.2696x over previous
import jax
import jax.numpy as jnp
from jax import lax
from jax.experimental import pallas as pl
from jax.experimental.pallas import tpu as pltpu

N_DEV = 32
M_PER = 1024
K = 512
N_OUT = 512

M_STREAM = 256
STREAM_OPS = [
    ["x", "y4", "z4"],
    ["y4", "z4", "x"],
    ["z4", "x", "y4"],
    ["x", "z4", "y4"],
]
N_ROUNDS = 3
N_SEMS = 64


def kernel(t, W):
    def body(t_ref, w_ref, out_ref, acc_ref, rbuf_ref, send_sems, recv_sems):
        my = lax.axis_index("i")
        zc = my >> 3
        msub = my & 7
        yc = msub >> 1
        xc = (msub ^ yc) & 1

        def y_partner(cc):
            return 8 * zc + 2 * cc + (xc ^ (cc & 1))

        def z_partner(cc):
            return 8 * cc + msub

        partners = [my ^ 1]
        partners += [y_partner((yc + k) & 3) for k in range(1, 4)]
        partners += [z_partner((zc + k) & 3) for k in range(1, 4)]

        barrier_sem = pltpu.get_barrier_semaphore()
        for p in partners:
            pl.semaphore_signal(
                barrier_sem, inc=1,
                device_id=(p,), device_id_type=pl.DeviceIdType.MESH,
            )
        pl.semaphore_wait(barrier_sem, len(partners))

        acc_ref[:, :] = t_ref[:, :]

        ctr = [0]

        def make(src, dst, dev):
            i = ctr[0]
            ctr[0] += 1
            return pltpu.make_async_remote_copy(
                src_ref=src, dst_ref=dst,
                send_sem=send_sems.at[i], recv_sem=recv_sems.at[i],
                device_id=(dev,), device_id_type=pl.DeviceIdType.MESH,
            )

        n_s = len(STREAM_OPS)
        los = [jnp.int32(s * M_STREAM) for s in range(n_s)]
        ws = [M_STREAM] * n_s
        rb_offs = [s * 512 for s in range(n_s)]

        for r in range(N_ROUNDS):
            pend = []
            for s in range(n_s):
                op = STREAM_OPS[s][r]
                if op == "x":
                    half = ws[s] // 2
                    send_lo = los[s] + (1 - xc) * half
                    keep_lo = los[s] + xc * half
                    rd = make(
                        acc_ref.at[pl.ds(send_lo, half)],
                        rbuf_ref.at[pl.ds(rb_offs[s], half)],
                        my ^ 1,
                    )
                    rd.start()
                    pend.append((s, [rd], keep_lo, half, rb_offs[s], 1))
                    los[s] = keep_lo
                    ws[s] = half
                    rb_offs[s] += half
                else:
                    q = ws[s] // 4
                    c = yc if op == "y4" else zc
                    keep_lo = los[s] + c * q
                    rds = []
                    for k in range(1, 4):
                        cc = (c + k) & 3
                        dev = y_partner(cc) if op == "y4" else z_partner(cc)
                        rd = make(
                            acc_ref.at[pl.ds(los[s] + cc * q, q)],
                            rbuf_ref.at[pl.ds(rb_offs[s] + (k - 1) * q, q)],
                            dev,
                        )
                        rd.start()
                        rds.append(rd)
                    pend.append((s, rds, keep_lo, q, rb_offs[s], 3))
                    los[s] = keep_lo
                    ws[s] = q
                    rb_offs[s] += 3 * q
            for s, rds, keep_lo, q, off, n_in in pend:
                for rd in rds:
                    rd.wait()
                total = rbuf_ref[pl.ds(off, q), :]
                for j in range(1, n_in):
                    total = total + rbuf_ref[pl.ds(off + j * q, q), :]
                acc_ref[pl.ds(keep_lo, q), :] = (
                    acc_ref[pl.ds(keep_lo, q), :] + total
                )

        for s in range(n_s):
            out_ref[pl.ds(los[s], ws[s]), :] = jnp.dot(
                acc_ref[pl.ds(los[s], ws[s]), :], w_ref[:, :],
                preferred_element_type=jnp.float32,
            )

        for j in range(N_ROUNDS):
            pend = []
            for s in range(n_s):
                op = STREAM_OPS[s][N_ROUNDS - 1 - j]
                w = ws[s]
                src = out_ref.at[pl.ds(los[s], w)]
                if op == "x":
                    rd = make(src, out_ref.at[pl.ds(los[s], w)], my ^ 1)
                    rd.start()
                    pend.append((s, [rd], xc, 2))
                else:
                    c = yc if op == "y4" else zc
                    rds = []
                    for k in range(1, 4):
                        cc = (c + k) & 3
                        dev = y_partner(cc) if op == "y4" else z_partner(cc)
                        rd = make(src, out_ref.at[pl.ds(los[s], w)], dev)
                        rd.start()
                        rds.append(rd)
                    pend.append((s, rds, c, 4))
            for s, rds, c, radix in pend:
                for rd in rds:
                    rd.wait()
                los[s] = los[s] - c * ws[s]
                ws[s] = ws[s] * radix

    return pl.pallas_call(
        body,
        out_shape=jax.ShapeDtypeStruct((M_PER, N_OUT), jnp.float32),
        in_specs=[
            pl.BlockSpec(memory_space=pltpu.VMEM),
            pl.BlockSpec(memory_space=pltpu.VMEM),
        ],
        out_specs=pl.BlockSpec(memory_space=pltpu.VMEM),
        scratch_shapes=[
            pltpu.VMEM((M_PER, K), jnp.float32),
            pltpu.VMEM((4 * 512, K), jnp.float32),
            pltpu.SemaphoreType.DMA((N_SEMS,)),
            pltpu.SemaphoreType.DMA((N_SEMS,)),
        ],
        compiler_params=pltpu.CompilerParams(collective_id=0),
    )(t, W)


# device time: 42512 ns/iter; 1.7166x vs baseline; 1.0395x over previous
import jax
import jax.numpy as jnp
from jax import lax
from jax.experimental import pallas as pl
from jax.experimental.pallas import tpu as pltpu

N_DEV = 32
M_PER = 1024
K = 512
N_OUT = 512

M_STREAM = 256
STREAM_OPS = [
    ["x", "y4", "z4"],
    ["y4", "z4", "x"],
    ["z4", "x", "y4"],
    ["x", "z4", "y4"],
]
N_ROUNDS = 3
N_SEMS = 64


def kernel(t, W):
    def body(t_ref, w_ref, out_ref, acc_ref, rbuf_ref, send_sems, recv_sems):
        my = lax.axis_index("i")
        zc = my >> 3
        msub = my & 7
        yc = msub >> 1
        xc = (msub ^ yc) & 1

        def y_partner(cc):
            return 8 * zc + 2 * cc + (xc ^ (cc & 1))

        def z_partner(cc):
            return 8 * cc + msub

        partners = [my ^ 1]
        partners += [y_partner((yc + k) & 3) for k in range(1, 4)]
        partners += [z_partner((zc + k) & 3) for k in range(1, 4)]

        barrier_sem = pltpu.get_barrier_semaphore()
        for p in partners:
            pl.semaphore_signal(
                barrier_sem, inc=1,
                device_id=(p,), device_id_type=pl.DeviceIdType.MESH,
            )
        pl.semaphore_wait(barrier_sem, len(partners))

        ctr = [0]

        def make(src, dst, dev):
            i = ctr[0]
            ctr[0] += 1
            return pltpu.make_async_remote_copy(
                src_ref=src, dst_ref=dst,
                send_sem=send_sems.at[i], recv_sem=recv_sems.at[i],
                device_id=(dev,), device_id_type=pl.DeviceIdType.MESH,
            )

        n_s = len(STREAM_OPS)
        los = [jnp.int32(s * M_STREAM) for s in range(n_s)]
        ws = [M_STREAM] * n_s
        rb_offs = [s * 512 for s in range(n_s)]
        pend = [None] * n_s

        def issue_rs(s, r):
            src_buf = t_ref if r == 0 else acc_ref
            op = STREAM_OPS[s][r]
            if op == "x":
                half = ws[s] // 2
                send_lo = los[s] + (1 - xc) * half
                keep_lo = los[s] + xc * half
                rd = make(
                    src_buf.at[pl.ds(send_lo, half)],
                    rbuf_ref.at[pl.ds(rb_offs[s], half)],
                    my ^ 1,
                )
                rd.start()
                pend[s] = ("rs", r, [rd], keep_lo, half, rb_offs[s], 1)
                los[s] = keep_lo
                ws[s] = half
                rb_offs[s] += half
            else:
                q = ws[s] // 4
                c = yc if op == "y4" else zc
                keep_lo = los[s] + c * q
                rds = []
                for k in range(1, 4):
                    cc = (c + k) & 3
                    dev = y_partner(cc) if op == "y4" else z_partner(cc)
                    rd = make(
                        src_buf.at[pl.ds(los[s] + cc * q, q)],
                        rbuf_ref.at[pl.ds(rb_offs[s] + (k - 1) * q, q)],
                        dev,
                    )
                    rd.start()
                    rds.append(rd)
                pend[s] = ("rs", r, rds, keep_lo, q, rb_offs[s], 3)
                los[s] = keep_lo
                ws[s] = q
                rb_offs[s] += 3 * q

        def issue_ag(s, j):
            op = STREAM_OPS[s][N_ROUNDS - 1 - j]
            w = ws[s]
            src = out_ref.at[pl.ds(los[s], w)]
            if op == "x":
                rd = make(src, out_ref.at[pl.ds(los[s], w)], my ^ 1)
                rd.start()
                pend[s] = ("ag", [rd], xc, 2)
            else:
                c = yc if op == "y4" else zc
                rds = []
                for k in range(1, 4):
                    cc = (c + k) & 3
                    dev = y_partner(cc) if op == "y4" else z_partner(cc)
                    rd = make(src, out_ref.at[pl.ds(los[s], w)], dev)
                    rd.start()
                    rds.append(rd)
                pend[s] = ("ag", rds, c, 4)

        def finish(s):
            if pend[s] is None:
                return
            if pend[s][0] == "rs":
                _, r, rds, keep_lo, q, off, n_in = pend[s]
                for rd in rds:
                    rd.wait()
                base_buf = t_ref if r == 0 else acc_ref
                total = rbuf_ref[pl.ds(off, q), :]
                for j in range(1, n_in):
                    total = total + rbuf_ref[pl.ds(off + j * q, q), :]
                acc_ref[pl.ds(keep_lo, q), :] = (
                    base_buf[pl.ds(keep_lo, q), :] + total
                )
            else:
                _, rds, c, radix = pend[s]
                for rd in rds:
                    rd.wait()
                los[s] = los[s] - c * ws[s]
                ws[s] = ws[s] * radix
            pend[s] = None

        for step in range(2 * N_ROUNDS + 1):
            for s in range(n_s):
                finish(s)
                if step < N_ROUNDS:
                    issue_rs(s, step)
                elif step == N_ROUNDS:
                    out_ref[pl.ds(los[s], ws[s]), :] = jnp.dot(
                        acc_ref[pl.ds(los[s], ws[s]), :], w_ref[:, :],
                        preferred_element_type=jnp.float32,
                    )
                else:
                    issue_ag(s, step - N_ROUNDS - 1)
        for s in range(n_s):
            finish(s)

    return pl.pallas_call(
        body,
        out_shape=jax.ShapeDtypeStruct((M_PER, N_OUT), jnp.float32),
        in_specs=[
            pl.BlockSpec(memory_space=pltpu.VMEM),
            pl.BlockSpec(memory_space=pltpu.VMEM),
        ],
        out_specs=pl.BlockSpec(memory_space=pltpu.VMEM),
        scratch_shapes=[
            pltpu.VMEM((M_PER, K), jnp.float32),
            pltpu.VMEM((4 * 512, K), jnp.float32),
            pltpu.SemaphoreType.DMA((N_SEMS,)),
            pltpu.SemaphoreType.DMA((N_SEMS,)),
        ],
        compiler_params=pltpu.CompilerParams(collective_id=0),
    )(t, W)
